# baseline (speedup 1.0000x reference)
import functools

import jax
import jax.numpy as jnp
from jax import lax
from jax.experimental import pallas as pl
from jax.experimental.pallas import tpu as pltpu
from jax.experimental.pallas import tpu_sc as plsc

NUM_CORES = 2
NUM_SUBCORES = 16
NUM_WORKERS = NUM_CORES * NUM_SUBCORES

LANES = 16
CHUNK = 128
NBUF = 2


def _make_gather(n_b: int, n_s: int, dim: int):
  n_st = n_s // 8
  total = n_b * n_s
  per_w = total // NUM_WORKERS
  assert per_w * NUM_WORKERS == total
  n_chunks = per_w // CHUNK
  assert n_chunks * CHUNK == per_w
  assert (n_chunks - NBUF) % NBUF == 0
  n_dt = dim // 8
  n_bt = n_b // CHUNK
  mesh = plsc.VectorSubcoreMesh(core_axis_name="c", subcore_axis_name="s")

  @functools.partial(
      pl.kernel,
      out_type=jax.ShapeDtypeStruct((n_s, n_dt, n_bt, 8, CHUNK), jnp.float32),
      mesh=mesh,
      scratch_types=[
          pltpu.VMEM((per_w,), jnp.int32),
          [pltpu.VMEM((CHUNK, dim), jnp.float32) for _ in range(NBUF)],
          [pltpu.VMEM((n_dt * 8, CHUNK + 1), jnp.float32) for _ in range(NBUF)],
          pltpu.SemaphoreType.DMA,
          [pltpu.SemaphoreType.DMA for _ in range(NBUF)],
          [pltpu.SemaphoreType.DMA for _ in range(NBUF)],
      ],
      compiler_params=pltpu.CompilerParams(
          use_tc_tiling_on_sc=False, needs_layout_passes=False),
  )
  def gather_kernel(idx_hbm, table_hbm, out_hbm, idx_v, rows, tiles,
                    isem, gsem, ssem):
    wid = lax.axis_index("s") * NUM_CORES + lax.axis_index("c")
    base = wid * per_w

    @pl.loop(0, n_chunks)
    def _(c):
      t0 = base + c * CHUNK
      s = t0 // n_b
      bt = (t0 % n_b) // CHUNK
      pltpu.async_copy(idx_hbm.at[s // 8, bt, s % 8],
                       idx_v.at[pl.ds(c * CHUNK, CHUNK)], isem)
    @pl.loop(0, n_chunks)
    def _(c):
      pltpu.make_async_copy(idx_hbm.at[0, 0, 0],
                            idx_v.at[pl.ds(0, CHUNK)], isem).wait()

    iota = lax.iota(jnp.int32, LANES)
    rvecs = [d0 + iota for d0 in range(0, dim, LANES)]

    def fire_gather(c, b):
      pltpu.async_copy(
          table_hbm.at[idx_v.at[pl.ds(c * CHUNK, CHUNK)]], rows[b], gsem[b])

    def wait_gather(b):
      pltpu.make_async_copy(
          table_hbm.at[pl.ds(0, CHUNK)], rows[b], gsem[b]).wait()

    def fire_store(c, b):
      t0 = base + c * CHUNK
      s = t0 // n_b
      bt = (t0 % n_b) // CHUNK
      for dt in range(n_dt):
        pltpu.async_copy(
            tiles[b].at[pl.ds(dt * 8, 8), pl.ds(0, CHUNK)],
            out_hbm.at[s, dt, bt], ssem[b])

    def wait_store(b):
      for dt in range(n_dt):
        pltpu.make_async_copy(
            tiles[b].at[pl.ds(dt * 8, 8), pl.ds(0, CHUNK)],
            out_hbm.at[0, 0, 0], ssem[b]).wait()

    def transpose(b):
      @plsc.parallel_loop(0, CHUNK, unroll=8)
      def _(bs):
        col = jnp.full((LANES,), bs, jnp.int32)
        for i, rvec in enumerate(rvecs):
          v = rows[b][bs, pl.ds(i * LANES, LANES)]
          plsc.store_scatter(tiles[b], [rvec, col], v)

    for b in range(NBUF):
      fire_gather(b, b)

    @pl.loop(0, n_chunks - NBUF, step=NBUF)
    def _(g0):
      for b in range(NBUF):
        wait_gather(b)
        transpose(b)
        fire_store(g0 + b, b)
        fire_gather(g0 + b + NBUF, b)
      for b in range(NBUF):
        wait_store(b)

    for b in range(NBUF):
      c = n_chunks - NBUF + b
      wait_gather(b)
      transpose(b)
      fire_store(c, b)
    for b in range(NBUF):
      wait_store(b)

  return gather_kernel


def kernel(input_ids, table):
  n_b, n_s = input_ids.shape
  dim = table.shape[1]
  ids4 = input_ids.T.reshape(n_s // 8, 8, n_b // CHUNK, CHUNK)
  ids4 = ids4.transpose(0, 2, 1, 3).astype(jnp.int32)
  table_p = lax.optimization_barrier(table.reshape(-1, 2 * dim))
  out5 = _make_gather(n_b, n_s, dim)(ids4, table_p.reshape(table.shape))
  out5 = out5.reshape(n_s, dim // 8, n_b // CHUNK, 8, CHUNK)
  return out5.transpose(2, 4, 0, 1, 3).reshape(n_b, n_s, dim)


# speedup vs baseline: 1.0036x; 1.0036x over previous
import functools

import jax
import jax.numpy as jnp
from jax import lax
from jax.experimental import pallas as pl
from jax.experimental.pallas import tpu as pltpu
from jax.experimental.pallas import tpu_sc as plsc

NUM_CORES = 2
NUM_SUBCORES = 16
NUM_WORKERS = NUM_CORES * NUM_SUBCORES

LANES = 16
CHUNK = 128
NBUF = 2


def _make_gather(n_b: int, n_s: int, dim: int):
  n_st = n_s // 8
  total = n_b * n_s
  per_w = total // NUM_WORKERS
  assert per_w * NUM_WORKERS == total
  n_chunks = per_w // CHUNK
  assert n_chunks * CHUNK == per_w
  assert (n_chunks - NBUF) % NBUF == 0
  n_dt = dim // 8
  n_bt = n_b // CHUNK
  mesh = plsc.VectorSubcoreMesh(core_axis_name="c", subcore_axis_name="s")

  @functools.partial(
      pl.kernel,
      out_type=jax.ShapeDtypeStruct((n_s, n_dt, n_bt, 8, CHUNK), jnp.float32),
      mesh=mesh,
      scratch_types=[
          pltpu.VMEM((per_w,), jnp.int32),
          [pltpu.VMEM((CHUNK, 2 * dim), jnp.float32) for _ in range(NBUF)],
          [pltpu.VMEM((n_dt * 8, CHUNK + 1), jnp.float32) for _ in range(NBUF)],
          pltpu.SemaphoreType.DMA,
          [pltpu.SemaphoreType.DMA for _ in range(NBUF)],
          [pltpu.SemaphoreType.DMA for _ in range(NBUF)],
      ],
      compiler_params=pltpu.CompilerParams(
          use_tc_tiling_on_sc=False, needs_layout_passes=False),
  )
  def gather_kernel(idx_hbm, table_hbm, out_hbm, idx_v, rows, tiles,
                    isem, gsem, ssem):
    wid = lax.axis_index("s") * NUM_CORES + lax.axis_index("c")
    base = wid * per_w

    @pl.loop(0, n_chunks)
    def _(c):
      t0 = base + c * CHUNK
      s = t0 // n_b
      bt = (t0 % n_b) // CHUNK
      pltpu.async_copy(idx_hbm.at[s // 8, bt, s % 8],
                       idx_v.at[pl.ds(c * CHUNK, CHUNK)], isem)
    @pl.loop(0, n_chunks)
    def _(c):
      pltpu.make_async_copy(idx_hbm.at[0, 0, 0],
                            idx_v.at[pl.ds(0, CHUNK)], isem).wait()

    iota = lax.iota(jnp.int32, LANES)
    rvecs = [d0 + iota for d0 in range(0, dim, LANES)]

    def fire_gather(c, b):
      pltpu.async_copy(
          table_hbm.at[idx_v.at[pl.ds(c * CHUNK, CHUNK)]], rows[b], gsem[b])

    def wait_gather(b):
      pltpu.make_async_copy(
          table_hbm.at[pl.ds(0, CHUNK)], rows[b], gsem[b]).wait()

    def fire_store(c, b):
      t0 = base + c * CHUNK
      s = t0 // n_b
      bt = (t0 % n_b) // CHUNK
      for dt in range(n_dt):
        pltpu.async_copy(
            tiles[b].at[pl.ds(dt * 8, 8), pl.ds(0, CHUNK)],
            out_hbm.at[s, dt, bt], ssem[b])

    def wait_store(b):
      for dt in range(n_dt):
        pltpu.make_async_copy(
            tiles[b].at[pl.ds(dt * 8, 8), pl.ds(0, CHUNK)],
            out_hbm.at[0, 0, 0], ssem[b]).wait()

    def transpose(b):
      @plsc.parallel_loop(0, CHUNK, unroll=8)
      def _(bs):
        col = jnp.full((LANES,), bs, jnp.int32)
        for i, rvec in enumerate(rvecs):
          v = rows[b][bs, pl.ds(i * LANES, LANES)]
          plsc.store_scatter(tiles[b], [rvec, col], v)

    for b in range(NBUF):
      fire_gather(b, b)

    @pl.loop(0, n_chunks - NBUF, step=NBUF)
    def _(g0):
      for b in range(NBUF):
        wait_gather(b)
        transpose(b)
        fire_store(g0 + b, b)
        fire_gather(g0 + b + NBUF, b)
      for b in range(NBUF):
        wait_store(b)

    for b in range(NBUF):
      c = n_chunks - NBUF + b
      wait_gather(b)
      transpose(b)
      fire_store(c, b)
    for b in range(NBUF):
      wait_store(b)

  return gather_kernel


def kernel(input_ids, table):
  n_b, n_s = input_ids.shape
  dim = table.shape[1]
  ids4 = input_ids.T.reshape(n_s // 8, 8, n_b // CHUNK, CHUNK)
  ids4 = ids4.transpose(0, 2, 1, 3).astype(jnp.int32)
  table_w = jnp.pad(table, ((0, 0), (0, dim)))
  out5 = _make_gather(n_b, n_s, dim)(ids4, table_w)
  out5 = out5.reshape(n_s, dim // 8, n_b // CHUNK, 8, CHUNK)
  return out5.transpose(2, 4, 0, 1, 3).reshape(n_b, n_s, dim)


# speedup vs baseline: 1.0335x; 1.0298x over previous
import functools

import jax
import jax.numpy as jnp
from jax import lax
from jax.experimental import pallas as pl
from jax.experimental.pallas import tpu as pltpu
from jax.experimental.pallas import tpu_sc as plsc

NUM_CORES = 2
NUM_SUBCORES = 16
NUM_WORKERS = NUM_CORES * NUM_SUBCORES

LANES = 16
CHUNK = 128
NBUF = 4


def _make_gather(n_b: int, n_s: int, dim: int):
  n_st = n_s // 8
  total = n_b * n_s
  per_w = total // NUM_WORKERS
  assert per_w * NUM_WORKERS == total
  n_chunks = per_w // CHUNK
  assert n_chunks * CHUNK == per_w
  assert (n_chunks - NBUF) % NBUF == 0
  n_dt = dim // 8
  n_bt = n_b // CHUNK
  mesh = plsc.VectorSubcoreMesh(core_axis_name="c", subcore_axis_name="s")

  @functools.partial(
      pl.kernel,
      out_type=jax.ShapeDtypeStruct((n_s, n_dt, n_bt, 8, CHUNK), jnp.float32),
      mesh=mesh,
      scratch_types=[
          pltpu.VMEM((per_w,), jnp.int32),
          [pltpu.VMEM((CHUNK, 2 * dim), jnp.float32) for _ in range(NBUF)],
          [pltpu.VMEM((n_dt * 8, CHUNK + 1), jnp.float32) for _ in range(NBUF)],
          pltpu.SemaphoreType.DMA,
          [pltpu.SemaphoreType.DMA for _ in range(NBUF)],
          [pltpu.SemaphoreType.DMA for _ in range(NBUF)],
      ],
      compiler_params=pltpu.CompilerParams(
          use_tc_tiling_on_sc=False, needs_layout_passes=False),
  )
  def gather_kernel(idx_hbm, table_hbm, out_hbm, idx_v, rows, tiles,
                    isem, gsem, ssem):
    wid = lax.axis_index("s") * NUM_CORES + lax.axis_index("c")
    base = wid * per_w

    @pl.loop(0, n_chunks)
    def _(c):
      t0 = base + c * CHUNK
      s = t0 // n_b
      bt = (t0 % n_b) // CHUNK
      pltpu.async_copy(idx_hbm.at[s // 8, bt, s % 8],
                       idx_v.at[pl.ds(c * CHUNK, CHUNK)], isem)
    @pl.loop(0, n_chunks)
    def _(c):
      pltpu.make_async_copy(idx_hbm.at[0, 0, 0],
                            idx_v.at[pl.ds(0, CHUNK)], isem).wait()

    iota = lax.iota(jnp.int32, LANES)
    rvecs = [d0 + iota for d0 in range(0, dim, LANES)]

    def fire_gather(c, b):
      pltpu.async_copy(
          table_hbm.at[idx_v.at[pl.ds(c * CHUNK, CHUNK)]], rows[b], gsem[b])

    def wait_gather(b):
      pltpu.make_async_copy(
          table_hbm.at[pl.ds(0, CHUNK)], rows[b], gsem[b]).wait()

    def fire_store(c, b):
      t0 = base + c * CHUNK
      s = t0 // n_b
      bt = (t0 % n_b) // CHUNK
      for dt in range(n_dt):
        pltpu.async_copy(
            tiles[b].at[pl.ds(dt * 8, 8), pl.ds(0, CHUNK)],
            out_hbm.at[s, dt, bt], ssem[b])

    def wait_store(b):
      for dt in range(n_dt):
        pltpu.make_async_copy(
            tiles[b].at[pl.ds(dt * 8, 8), pl.ds(0, CHUNK)],
            out_hbm.at[0, 0, 0], ssem[b]).wait()

    def transpose(b):
      @plsc.parallel_loop(0, CHUNK, unroll=8)
      def _(bs):
        col = jnp.full((LANES,), bs, jnp.int32)
        for i, rvec in enumerate(rvecs):
          v = rows[b][bs, pl.ds(i * LANES, LANES)]
          plsc.store_scatter(tiles[b], [rvec, col], v)

    for b in range(NBUF):
      fire_gather(b, b)

    @pl.loop(0, n_chunks - NBUF, step=NBUF)
    def _(g0):
      for b in range(NBUF):
        wait_gather(b)
        transpose(b)
        fire_store(g0 + b, b)
        fire_gather(g0 + b + NBUF, b)
      for b in range(NBUF):
        wait_store(b)

    for b in range(NBUF):
      c = n_chunks - NBUF + b
      wait_gather(b)
      transpose(b)
      fire_store(c, b)
    for b in range(NBUF):
      wait_store(b)

  return gather_kernel


def kernel(input_ids, table):
  n_b, n_s = input_ids.shape
  dim = table.shape[1]
  ids4 = input_ids.T.reshape(n_s // 8, 8, n_b // CHUNK, CHUNK)
  ids4 = ids4.transpose(0, 2, 1, 3).astype(jnp.int32)
  table_w = jnp.pad(table, ((0, 0), (0, dim)))
  out5 = _make_gather(n_b, n_s, dim)(ids4, table_w)
  out5 = out5.reshape(n_s, dim // 8, n_b // CHUNK, 8, CHUNK)
  return out5.transpose(2, 4, 0, 1, 3).reshape(n_b, n_s, dim)
